# Initial kernel scaffold; baseline (speedup 1.0000x reference)
import jax
import jax.numpy as jnp
from jax.experimental import pallas as pl


def kernel(pos, x, lig_coord, W1, b1, gamma, beta, W2, b2):
    raise NotImplementedError("write your pallas kernel here")


# speedup vs baseline: 26.3133x; 26.3133x over previous
import functools

import jax
import jax.numpy as jnp
from jax import lax
from jax.experimental import pallas as pl
from jax.experimental.pallas import tpu as pltpu
from jax.experimental.pallas import tpu_sc as plsc

_B, _N, _D, _L, _OUT, _K = 8, 16384, 256, 50, 7, 10
_LP = 64
_KP = 16
_M = _LP * _KP
_TRASH = 15


def _topk_body(posT_ref, lig_ref, out_ref):
    pos = posT_ref[0]
    lig = lig_ref[0]
    d2 = jnp.zeros((_LP, _N), jnp.float32)
    for c in range(3):
        diff = lig[:, c:c + 1] - pos[c:c + 1, :]
        d2 = d2 + diff * diff
    ioN = lax.broadcasted_iota(jnp.int32, (_LP, _N), 1)
    liota = lax.broadcasted_iota(jnp.int32, (_LP, _KP), 0)
    kiota = lax.broadcasted_iota(jnp.int32, (_LP, _KP), 1)
    pad_val = _N + liota * _KP + kiota
    lvalid = liota < _L
    big = jnp.int32(2 ** 30)
    for k in range(_K):
        m = jnp.min(d2, axis=1, keepdims=True)
        cand = jnp.where(d2 == m, ioN, big)
        amin = jnp.min(cand, axis=1, keepdims=True)
        col = jnp.where(lvalid[:, k:k + 1], amin, pad_val[:, k:k + 1])
        out_ref[0, :, k:k + 1] = col
        d2 = jnp.where(ioN == amin, jnp.float32(jnp.inf), d2)
    for k in range(_K, _KP):
        out_ref[0, :, k:k + 1] = pad_val[:, k:k + 1]


def _sc_pool_body(idx_hbm, x_hbm, partial_hbm, cnt_hbm,
                  idx_v, tag_v, gidx_v, rows_v, acc_v, cnt_v, sem):
    c = lax.axis_index("c")
    s = lax.axis_index("s")
    wid = s * 2 + c
    b = wid // 4
    q = wid % 4

    pltpu.sync_copy(idx_hbm.at[b], idx_v)
    for j in range(_M // 16):
        v = idx_v[pl.ds(j * 16, 16)]
        slot = lax.iota(jnp.int32, 16) + j * 16
        plsc.store_scatter(tag_v, [v], slot)
    acc = jnp.zeros((16,), jnp.float32)
    for j in range(_M // 16):
        v = idx_v[pl.ds(j * 16, 16)]
        slot = lax.iota(jnp.int32, 16) + j * 16
        got = plsc.load_gather(tag_v, [v])
        keep = (got == slot) & (v < _N)
        acc = acc + jnp.where(keep, jnp.float32(1.0), jnp.float32(0.0))
        r, off = divmod(j * 16, 128)
        gidx_v[r, pl.ds(off, 16)] = b * _N + jnp.where(keep, v, 0)

    @pl.when(q == 0)
    def _store_cnt():
        cnt_v[...] = jnp.zeros((16,), jnp.float32) + jnp.sum(acc)
        pltpu.sync_copy(cnt_v, cnt_hbm.at[b])

    accs = [jnp.zeros((16,), jnp.float32) for _ in range(_D // 16)]
    for ch in range(2):
        pltpu.async_copy(x_hbm.at[gidx_v.at[q * 2 + ch]], rows_v, sem).wait()

        def body(j, carry):
            return tuple(a + rows_v[j, pl.ds(k * 16, 16)]
                         for k, a in enumerate(carry))

        accs = list(lax.fori_loop(0, 128, body, tuple(accs)))
    for k in range(_D // 16):
        acc_v[pl.ds(k * 16, 16)] = accs[k]
    pltpu.sync_copy(acc_v, partial_hbm.at[q * _B + b])


def _sc_pool_call(idx2, xflat):
    mesh = plsc.VectorSubcoreMesh(core_axis_name="c", subcore_axis_name="s")
    sc_pool = functools.partial(
        pl.kernel,
        mesh=mesh,
        compiler_params=pltpu.CompilerParams(
            needs_layout_passes=False, use_tc_tiling_on_sc=False),
        out_type=[
            jax.ShapeDtypeStruct((4 * _B, _D), jnp.float32),
            jax.ShapeDtypeStruct((_B, 16), jnp.float32),
        ],
        scratch_types=[
            pltpu.VMEM((_M,), jnp.int32),
            pltpu.VMEM((_N + _M,), jnp.int32),
            pltpu.VMEM((8, 128), jnp.int32),
            pltpu.VMEM((128, _D), jnp.float32),
            pltpu.VMEM((_D,), jnp.float32),
            pltpu.VMEM((16,), jnp.float32),
            pltpu.SemaphoreType.DMA,
        ],
    )(_sc_pool_body)
    return sc_pool(idx2, xflat)


def _mlp_body(p_ref, cnt_ref, xb0_ref, w1_ref, b1_ref, g_ref, be_ref, w2_ref,
              b2_ref, out_ref):
    cnt = cnt_ref[:, 0:1]
    total = p_ref[0] + p_ref[1] + p_ref[2] + p_ref[3]
    pockets = (total - (_M - cnt) * xb0_ref[...]) / cnt
    h = jnp.dot(pockets, w1_ref[...],
                preferred_element_type=jnp.float32) + b1_ref[...]
    mu = jnp.mean(h, axis=-1, keepdims=True)
    var = jnp.mean((h - mu) ** 2, axis=-1, keepdims=True)
    h = (h - mu) / jnp.sqrt(var + 1e-5) * g_ref[...] + be_ref[...]
    h = h * jax.nn.sigmoid(h)
    out_ref[...] = jnp.dot(h, w2_ref[...],
                           preferred_element_type=jnp.float32) + b2_ref[...]


def kernel(pos, x, lig_coord, W1, b1, gamma, beta, W2, b2):
    posT = jnp.transpose(pos, (0, 2, 1))
    ligp = jnp.zeros((_B, _LP, 3), jnp.float32).at[:, :_L].set(lig_coord)

    idx = pl.pallas_call(
        _topk_body,
        grid=(_B,),
        in_specs=[
            pl.BlockSpec((1, 3, _N), lambda b: (b, 0, 0)),
            pl.BlockSpec((1, _LP, 3), lambda b: (b, 0, 0)),
        ],
        out_specs=pl.BlockSpec((1, _LP, _KP), lambda b: (b, 0, 0)),
        out_shape=jax.ShapeDtypeStruct((_B, _LP, _KP), jnp.int32),
    )(posT, ligp)

    partial, cnt = _sc_pool_call(idx.reshape(_B, _M), x.reshape(_B * _N, _D))

    out = pl.pallas_call(
        _mlp_body,
        out_shape=jax.ShapeDtypeStruct((_B, _OUT), jnp.float32),
    )(partial.reshape(4, _B, _D), cnt, x[:, 0, :], W1, b1.reshape(1, _D),
      gamma.reshape(1, _D), beta.reshape(1, _D), W2, b2.reshape(1, _OUT))
    return out
